# baseline (speedup 1.0000x reference)
import functools

import jax
import jax.numpy as jnp
from jax import lax
from jax.experimental import pallas as pl
from jax.experimental.pallas import tpu as pltpu
from jax.experimental.pallas import tpu_sc as plsc

N = 10000
E = 320000
D = 128
H = 128
C = 40
BN_EPS = 1e-5

NC = 2
NS = 16
NW = NC * NS

ROW = 64
SB = 8
SLOTS = 5
EP = 327680
R = EP // ROW
NSB = R // SB
ITERS = NSB // NW
NP = 10240
RPS = NP // NS

_mesh = plsc.VectorSubcoreMesh(core_axis_name="c", subcore_axis_name="s")


@functools.partial(
    pl.kernel,
    mesh=_mesh,
    out_type=jax.ShapeDtypeStruct((NC, NP, D), jnp.float32),
    scratch_types=[
        pltpu.VMEM((SB, ROW), jnp.int32),
        pltpu.VMEM((SB, ROW), jnp.int32),
        pltpu.VMEM((SLOTS * ROW, D), jnp.float32),
        pltpu.VMEM_SHARED((NP, D), jnp.float32),
        pltpu.SemaphoreType.DMA((SLOTS,)),
        pltpu.SemaphoreType.DMA((SLOTS,)),
    ],
)
def _segsum_sc(x_hbm, src_hbm, dst_hbm, zeros_hbm, out_hbm,
               sidx, didx, rows, acc, gsem, ssem):
    cid = lax.axis_index("c")
    sid = lax.axis_index("s")
    wid = sid * NC + cid

    pltpu.sync_copy(zeros_hbm, acc.at[pl.ds(sid * RPS, RPS)])
    plsc.subcore_barrier()

    @pl.loop(0, ITERS)
    def _(i):
        base = (wid + NW * i) * SB
        pltpu.sync_copy(src_hbm.at[pl.ds(base, SB)], sidx)
        pltpu.sync_copy(dst_hbm.at[pl.ds(base, SB)], didx)

        def slot(j):
            return rows.at[pl.ds((j % SLOTS) * ROW, ROW)]

        gd = [None] * SB
        sd = [None] * SB
        for j in range(SLOTS):
            gd[j] = pltpu.async_copy(x_hbm.at[sidx.at[j]], slot(j),
                                     gsem.at[j % SLOTS])
        for j in range(SB):
            gd[j].wait()
            sd[j] = pltpu.async_copy(slot(j), acc.at[didx.at[j]],
                                     ssem.at[j % SLOTS], add=True)
            if j + SLOTS < SB:
                sd[j].wait()
                gd[j + SLOTS] = pltpu.async_copy(
                    x_hbm.at[sidx.at[j + SLOTS]], slot(j + SLOTS),
                    gsem.at[j % SLOTS])
        for j in range(SB - SLOTS, SB):
            sd[j].wait()

    plsc.subcore_barrier()

    pltpu.sync_copy(acc.at[pl.ds(sid * RPS, RPS)],
                    out_hbm.at[cid].at[pl.ds(sid * RPS, RPS)])


def _mlp_bn_relu_body(x_ref, p_ref, w_ref, b_ref, g_ref, be_ref, o_ref):
    h = x_ref[...] + p_ref[0][:N] + p_ref[1][:N]
    z = lax.dot_general(h, w_ref[...], (((1,), (1,)), ((), ())),
                        precision=lax.Precision.HIGHEST,
                        preferred_element_type=jnp.float32)
    z = z + b_ref[...]
    mu = jnp.mean(z, axis=0, keepdims=True)
    zc = z - mu
    var = jnp.mean(zc * zc, axis=0, keepdims=True)
    zn = g_ref[...] * (zc * lax.rsqrt(var + BN_EPS)) + be_ref[...]
    o_ref[...] = jnp.maximum(zn, 0.0)


def _final_body(x_ref, p_ref, w_ref, b_ref, o_ref):
    h = x_ref[...] + p_ref[0][:N] + p_ref[1][:N]
    z = lax.dot_general(h, w_ref[...], (((1,), (1,)), ((), ())),
                        precision=lax.Precision.HIGHEST,
                        preferred_element_type=jnp.float32)
    o_ref[...] = z + b_ref[...]


_mlp_bn_relu = pl.pallas_call(
    _mlp_bn_relu_body,
    out_shape=jax.ShapeDtypeStruct((N, H), jnp.float32),
)

_final = pl.pallas_call(
    _final_body,
    out_shape=jax.ShapeDtypeStruct((N, C), jnp.float32),
)


def kernel(x, edge_index, W1, b1, gamma1, beta1, W2, b2, gamma2, beta2,
           W3, b3):
    pad = EP - E
    src = jnp.concatenate(
        [edge_index[0], jnp.zeros((pad,), jnp.int32)]).reshape(R, ROW)
    dst = jnp.concatenate(
        [edge_index[1], jnp.full((pad,), NP - 1, jnp.int32)]).reshape(R, ROW)
    zeros = jnp.zeros((RPS, D), dtype=jnp.float32)

    p = _segsum_sc(x, src, dst, zeros)
    h = _mlp_bn_relu(x, p, W1, b1.reshape(1, H), gamma1.reshape(1, H),
                     beta1.reshape(1, H))
    p = _segsum_sc(h, src, dst, zeros)
    h = _mlp_bn_relu(h, p, W2, b2.reshape(1, H), gamma2.reshape(1, H),
                     beta2.reshape(1, H))
    p = _segsum_sc(h, src, dst, zeros)
    out = _final(h, p, W3, b3.reshape(1, C))
    return out


# speedup vs baseline: 1.0559x; 1.0559x over previous
import functools

import jax
import jax.numpy as jnp
from jax import lax
from jax.experimental import pallas as pl
from jax.experimental.pallas import tpu as pltpu
from jax.experimental.pallas import tpu_sc as plsc

N = 10000
E = 320000
D = 128
H = 128
C = 40
BN_EPS = 1e-5

NC = 2
NS = 16
NW = NC * NS

ROW = 128
SB = 16
HSB = SB // 2
EP = 327680
R = EP // ROW
NSB = R // SB
ITERS = NSB // NW
NP = 10240
RPS = NP // NS

_mesh = plsc.VectorSubcoreMesh(core_axis_name="c", subcore_axis_name="s")


@functools.partial(
    pl.kernel,
    mesh=_mesh,
    out_type=jax.ShapeDtypeStruct((NC, NP, D), jnp.float32),
    scratch_types=[
        pltpu.VMEM((SB, ROW), jnp.int32),
        pltpu.VMEM((SB, ROW), jnp.int32),
        pltpu.VMEM((2 * ROW, D), jnp.float32),
        pltpu.VMEM_SHARED((NP, D), jnp.float32),
        pltpu.SemaphoreType.DMA,
        pltpu.SemaphoreType.DMA,
        pltpu.SemaphoreType.DMA,
        pltpu.SemaphoreType.DMA,
    ],
)
def _segsum_sc(x_hbm, src_hbm, dst_hbm, zeros_hbm, out_hbm,
               sidx, didx, rows, acc, gsem0, gsem1, ssem0, ssem1):
    cid = lax.axis_index("c")
    sid = lax.axis_index("s")
    wid = sid * NC + cid
    gsems = (gsem0, gsem1)
    ssems = (ssem0, ssem1)

    pltpu.sync_copy(zeros_hbm, acc.at[pl.ds(sid * RPS, RPS)])
    plsc.subcore_barrier()

    @pl.loop(0, ITERS)
    def _(i):
        base = (wid + NW * i) * SB
        pltpu.sync_copy(src_hbm.at[pl.ds(base, SB)], sidx)
        pltpu.sync_copy(dst_hbm.at[pl.ds(base, SB)], didx)

        def slot(j):
            return rows.at[pl.ds((j % 2) * ROW, ROW)]

        @pl.loop(0, 2)
        def _(h):
            b = h * HSB
            gd = [None] * HSB
            sd = [None] * HSB
            for j in range(2):
                gd[j] = pltpu.async_copy(x_hbm.at[sidx.at[b + j]], slot(j),
                                         gsems[j % 2])
            for j in range(HSB):
                gd[j].wait()
                sd[j] = pltpu.async_copy(slot(j), acc.at[didx.at[b + j]],
                                         ssems[j % 2], add=True)
                if j + 2 < HSB:
                    sd[j].wait()
                    gd[j + 2] = pltpu.async_copy(
                        x_hbm.at[sidx.at[b + j + 2]], slot(j + 2),
                        gsems[j % 2])
            sd[HSB - 2].wait()
            sd[HSB - 1].wait()

    plsc.subcore_barrier()

    pltpu.sync_copy(acc.at[pl.ds(sid * RPS, RPS)],
                    out_hbm.at[cid].at[pl.ds(sid * RPS, RPS)])


def _mlp_bn_relu_body(x_ref, p_ref, w_ref, b_ref, g_ref, be_ref, o_ref):
    h = x_ref[...] + p_ref[0][:N] + p_ref[1][:N]
    z = lax.dot_general(h, w_ref[...], (((1,), (1,)), ((), ())),
                        precision=lax.Precision.HIGHEST,
                        preferred_element_type=jnp.float32)
    z = z + b_ref[...]
    mu = jnp.mean(z, axis=0, keepdims=True)
    zc = z - mu
    var = jnp.mean(zc * zc, axis=0, keepdims=True)
    zn = g_ref[...] * (zc * lax.rsqrt(var + BN_EPS)) + be_ref[...]
    o_ref[...] = jnp.maximum(zn, 0.0)


def _final_body(x_ref, p_ref, w_ref, b_ref, o_ref):
    h = x_ref[...] + p_ref[0][:N] + p_ref[1][:N]
    z = lax.dot_general(h, w_ref[...], (((1,), (1,)), ((), ())),
                        precision=lax.Precision.HIGHEST,
                        preferred_element_type=jnp.float32)
    o_ref[...] = z + b_ref[...]


_mlp_bn_relu = pl.pallas_call(
    _mlp_bn_relu_body,
    out_shape=jax.ShapeDtypeStruct((N, H), jnp.float32),
)

_final = pl.pallas_call(
    _final_body,
    out_shape=jax.ShapeDtypeStruct((N, C), jnp.float32),
)


def kernel(x, edge_index, W1, b1, gamma1, beta1, W2, b2, gamma2, beta2,
           W3, b3):
    pad = EP - E
    src = jnp.concatenate(
        [edge_index[0], jnp.zeros((pad,), jnp.int32)]).reshape(R, ROW)
    dst = jnp.concatenate(
        [edge_index[1], jnp.full((pad,), NP - 1, jnp.int32)]).reshape(R, ROW)
    zeros = jnp.zeros((RPS, D), dtype=jnp.float32)

    p = _segsum_sc(x, src, dst, zeros)
    h = _mlp_bn_relu(x, p, W1, b1.reshape(1, H), gamma1.reshape(1, H),
                     beta1.reshape(1, H))
    p = _segsum_sc(h, src, dst, zeros)
    h = _mlp_bn_relu(h, p, W2, b2.reshape(1, H), gamma2.reshape(1, H),
                     beta2.reshape(1, H))
    p = _segsum_sc(h, src, dst, zeros)
    out = _final(h, p, W3, b3.reshape(1, C))
    return out
